# baseline (speedup 1.0000x reference)
import functools

import jax
import jax.numpy as jnp
from jax import lax
from jax.experimental import pallas as pl
from jax.experimental.pallas import tpu as pltpu
from jax.experimental.pallas import tpu_sc as plsc

LANES = 128


def _router_body(x_ref, gW_ref, gb_ref, oh_ref, pos_ref, cnt_ref, x16_ref,
                 idx_ref, oh_all, run_s, base_s,
                 *, num_tiles, num_t, bt_blk, bmm):
    p = pl.program_id(0)
    t = pl.program_id(1)
    bt = oh_ref.shape[0]
    hp = jax.lax.Precision.HIGHEST
    ch = x_ref.shape[1] // 2
    xl = x_ref[:, :ch].astype(jnp.bfloat16).astype(jnp.float32)
    xr = x_ref[:, ch:].astype(jnp.bfloat16).astype(jnp.float32)
    ul = jax.lax.bitcast_convert_type(xl, jnp.uint32)
    ur = jax.lax.bitcast_convert_type(xr, jnp.uint32)
    w = (ul & jnp.uint32(0xFFFF0000)) | (ur >> 16)
    x16_ref[...] = jax.lax.bitcast_convert_type(w, jnp.float32)

    @pl.when(p == 0)
    def _pass0():
        @pl.when(t == 0)
        def _init():
            run_s[...] = jnp.zeros_like(run_s)

        logits = jax.lax.dot_general(
            x_ref[...], gW_ref[...], (((1,), (1,)), ((), ())),
            preferred_element_type=jnp.float32) + gb_ref[...]
        cols = jax.lax.broadcasted_iota(jnp.int32, (bt, LANES), 1)
        logits = jnp.where(cols < num_tiles, logits, jnp.float32(-3e38))
        m = jnp.max(logits, axis=1, keepdims=True)
        first = jnp.min(jnp.where(logits >= m, cols, jnp.int32(LANES)),
                        axis=1, keepdims=True)
        oh = (cols == first).astype(jnp.float32)
        oh_ref[...] = oh
        idx_ref[...] = jnp.broadcast_to(first, (bt, LANES))
        pos_ref[...] = jnp.zeros((bt, LANES), jnp.int32)
        oh_all[pl.ds(t * bt_blk, bt_blk), :] = oh
        run_s[...] += jnp.sum(oh, axis=0, keepdims=True)

        @pl.when(t == num_t - 1)
        def _fin():
            cnt = run_s[...].astype(jnp.int32)
            cnt_ref[...] = cnt
            capt = (((cnt + (bmm - 1)) // bmm) * bmm).astype(jnp.float32)
            r = jax.lax.broadcasted_iota(jnp.int32, (LANES, LANES), 0)
            c = jax.lax.broadcasted_iota(jnp.int32, (LANES, LANES), 1)
            triu = (r < c).astype(jnp.float32)
            base_s[...] = jax.lax.dot_general(
                capt, triu, (((1,), (0,)), ((), ())),
                preferred_element_type=jnp.float32, precision=hp)

    @pl.when(p == 1)
    def _pass1():
        @pl.when(t == 0)
        def _init():
            run_s[...] = base_s[...]

        oh = oh_all[pl.ds(t * bt_blk, bt_blk), :]
        oh_ref[...] = oh
        cols1 = jax.lax.broadcasted_iota(jnp.int32, (bt, LANES), 1)
        idx_ref[...] = jnp.broadcast_to(
            jnp.sum(oh * cols1.astype(jnp.float32), axis=1,
                    keepdims=True).astype(jnp.int32), (bt, LANES))
        r = jax.lax.broadcasted_iota(jnp.int32, (bt, bt), 0)
        c = jax.lax.broadcasted_iota(jnp.int32, (bt, bt), 1)
        tri = (r > c).astype(jnp.float32)
        ecs = jax.lax.dot_general(tri, oh, (((1,), (0,)), ((), ())),
                                  preferred_element_type=jnp.float32,
                                  precision=hp)
        pos = jnp.sum((ecs + run_s[...]) * oh, axis=1, keepdims=True)
        pos_ref[...] = jnp.broadcast_to(pos.astype(jnp.int32), (bt, LANES))
        run_s[...] += jnp.sum(oh, axis=0, keepdims=True)


def _router(xf, gate_W, gate_b, num_tiles, bmm, t_off=0, n_half=None):
    C = xf.shape[1]
    N = n_half if n_half is not None else xf.shape[0]
    BT = 512
    num_t = N // BT
    gW = jnp.zeros((LANES, C), jnp.float32).at[:num_tiles].set(gate_W)
    gb = jnp.zeros((1, LANES), jnp.float32).at[0, :num_tiles].set(gate_b)
    return pl.pallas_call(
        functools.partial(_router_body, num_tiles=num_tiles, num_t=num_t,
                          bt_blk=BT, bmm=bmm),
        grid=(2, num_t),
        in_specs=[
            pl.BlockSpec((BT, C), lambda p, t: (t_off + t * (1 - p), 0)),
            pl.BlockSpec((LANES, C), lambda p, t: (0, 0)),
            pl.BlockSpec((1, LANES), lambda p, t: (0, 0)),
        ],
        out_specs=[
            pl.BlockSpec((BT, LANES), lambda p, t: (t, 0)),
            pl.BlockSpec((BT, LANES), lambda p, t: (t, 0)),
            pl.BlockSpec((1, LANES), lambda p, t: (0, 0)),
            pl.BlockSpec((BT, C // 2), lambda p, t: (t * (1 - p), 0)),
            pl.BlockSpec((BT, LANES), lambda p, t: (t, 0)),
        ],
        out_shape=[
            jax.ShapeDtypeStruct((N, LANES), jnp.float32),
            jax.ShapeDtypeStruct((N, LANES), jnp.int32),
            jax.ShapeDtypeStruct((1, LANES), jnp.int32),
            jax.ShapeDtypeStruct((N, C // 2), jnp.float32),
            jax.ShapeDtypeStruct((N, LANES), jnp.int32),
        ],
        scratch_shapes=[
            pltpu.VMEM((N, LANES), jnp.float32),
            pltpu.VMEM((1, LANES), jnp.float32),
            pltpu.VMEM((1, LANES), jnp.float32),
        ],
    )(xf, gW, gb)


def _mm_body(bexp_ref, xs_ref, upW_ref, upb_ref, dW_ref, db_ref, y_ref,
             *, out_tile, nblk):
    i = pl.program_id(0)
    e = bexp_ref[i]
    used = bexp_ref[nblk]

    @pl.when(i < used)
    def _compute():
        u = jax.lax.bitcast_convert_type(xs_ref[...], jnp.uint32)
        xl = jax.lax.bitcast_convert_type(u & jnp.uint32(0xFFFF0000),
                                          jnp.float32)
        xr = jax.lax.bitcast_convert_type(u << 16, jnp.float32)
        x = jnp.concatenate([xl, xr], axis=1).astype(jnp.bfloat16)
        h = jax.lax.dot_general(x, upW_ref[0].astype(jnp.bfloat16),
                                (((1,), (1,)), ((), ())),
                                preferred_element_type=jnp.float32)
        h = jnp.maximum(h + upb_ref[0], 0.0).astype(jnp.bfloat16)
        y = jax.lax.dot_general(h, dW_ref[0].astype(jnp.bfloat16),
                                (((1,), (1,)), ((), ())),
                                preferred_element_type=jnp.float32)
        y_ref[...] = y + db_ref[0]


def _grouped_mm(xs, up_W, up_b, down_W, down_b, bexp, num_tiles, bmm):
    PAD_N = xs.shape[0]
    C = up_W.shape[1]
    d_ff = up_W.shape[0]
    ftile = d_ff // num_tiles
    out_tile = C // num_tiles
    nblk = PAD_N // bmm
    upW4 = up_W.reshape(num_tiles, ftile, C)
    upb3 = up_b.reshape(num_tiles, 1, ftile)
    dW4 = down_W.reshape(num_tiles, out_tile, num_tiles * ftile)
    db3 = down_b.reshape(num_tiles, 1, out_tile)
    grid_spec = pltpu.PrefetchScalarGridSpec(
        num_scalar_prefetch=1,
        grid=(nblk,),
        in_specs=[
            pl.BlockSpec((bmm, C // 2), lambda i, b: (i, 0)),
            pl.BlockSpec((1, ftile, C), lambda i, b: (b[i], 0, 0)),
            pl.BlockSpec((1, 1, ftile), lambda i, b: (b[i], 0, 0)),
            pl.BlockSpec((1, out_tile, ftile),
                         lambda i, b: (b[i], 0, b[i])),
            pl.BlockSpec((1, 1, out_tile), lambda i, b: (b[i], 0, 0)),
        ],
        out_specs=pl.BlockSpec((bmm, out_tile), lambda i, b: (i, 0)),
    )
    return pl.pallas_call(
        functools.partial(_mm_body, out_tile=out_tile, nblk=nblk),
        grid_spec=grid_spec,
        out_shape=jax.ShapeDtypeStruct((PAD_N, out_tile), jnp.float32),
    )(bexp, xs, upW4, upb3, dW4, db3)


def _expand_body(yc_ref, idx_ref, out_ref, *, out_tile):
    y = yc_ref[...]
    e = idx_ref[:, :1]
    reps = out_ref.shape[1] // out_tile
    ytile = jnp.concatenate([y] * reps, axis=1)
    ocols = jax.lax.broadcasted_iota(jnp.int32, ytile.shape, 1)
    out_ref[...] = jnp.where((ocols // out_tile) == e, ytile, 0.0)


def _expand_body_alias(yc_ref, idx_ref, prev_ref, out_ref, *, out_tile):
    del prev_ref
    _expand_body(yc_ref, idx_ref, out_ref, out_tile=out_tile)


def _expand(yc, idx2, C, blk_off=0, n_full=None, prev=None):
    N, out_tile = yc.shape
    n_full = n_full if n_full is not None else N
    BT = 1024
    in_specs = [
        pl.BlockSpec((BT, out_tile), lambda t: (t, 0)),
        pl.BlockSpec((BT, LANES), lambda t: (t, 0)),
    ]
    args = [yc, idx2]
    kwargs = {}
    body = functools.partial(_expand_body, out_tile=out_tile)
    if prev is not None:
        in_specs.append(pl.BlockSpec(memory_space=pltpu.HBM))
        args.append(prev)
        kwargs["input_output_aliases"] = {2: 0}
        body = functools.partial(_expand_body_alias, out_tile=out_tile)
    return pl.pallas_call(
        body,
        grid=(N // BT,),
        in_specs=in_specs,
        out_specs=pl.BlockSpec((BT, C), lambda t: (blk_off + t, 0)),
        out_shape=jax.ShapeDtypeStruct((n_full, C), jnp.float32),
        **kwargs,
    )(*args)


def _sc_mesh():
    info = plsc.get_sparse_core_info()
    return plsc.VectorSubcoreMesh(core_axis_name="c", subcore_axis_name="s"), \
        info.num_cores, info.num_subcores


def _dispatch(xf, pos, pad_n):
    N, C = xf.shape
    mesh, nc, ns = _sc_mesh()
    per_w = N // (nc * ns)
    CH = 64
    nchunk = per_w // CH

    @functools.partial(
        pl.kernel, mesh=mesh,
        out_type=jax.ShapeDtypeStruct((pad_n, C), xf.dtype),
        scratch_types=[
            pltpu.VMEM((CH,), jnp.int32),
            pltpu.VMEM((CH, C), xf.dtype),
            pltpu.SemaphoreType.DMA,
        ],
    )
    def k(xf_h, pos_h, xs_h, pos_v, xbuf, sem):
        wid = lax.axis_index("s") * nc + lax.axis_index("c")

        def chunk(j, _):
            n0 = wid * per_w + j * CH
            pltpu.sync_copy(pos_h.at[pl.ds(n0, CH)], pos_v)
            pltpu.sync_copy(xf_h.at[pl.ds(n0, CH)], xbuf)
            pltpu.async_copy(xbuf, xs_h.at[pos_v], sem).wait()
            return ()

        lax.fori_loop(0, nchunk, chunk, (), unroll=False)

    return k(xf, pos)


def _combine(y_full, pos, n_out):
    PAD_N, C = y_full.shape
    mesh, nc, ns = _sc_mesh()
    per_w = n_out // (nc * ns)
    CH = 128
    nchunk = per_w // CH

    @functools.partial(
        pl.kernel, mesh=mesh,
        out_type=jax.ShapeDtypeStruct((n_out, C), jnp.float32),
        scratch_types=[
            pltpu.VMEM((CH,), jnp.int32),
            pltpu.VMEM((CH, C), jnp.float32),
            pltpu.SemaphoreType.DMA,
        ],
    )
    def k(y_h, pos_h, out_h, pos_v, ybuf, sem):
        wid = lax.axis_index("s") * nc + lax.axis_index("c")

        def chunk(j, _):
            n0 = wid * per_w + j * CH
            pltpu.sync_copy(pos_h.at[pl.ds(n0, CH)], pos_v)
            pltpu.async_copy(y_h.at[pos_v], ybuf, sem).wait()
            pltpu.sync_copy(ybuf, out_h.at[pl.ds(n0, CH)])
            return ()

        lax.fori_loop(0, nchunk, chunk, (), unroll=False)

    return k(y_full, pos)


def _half(xf, up_W, up_b, down_W, down_b, gate_W, gate_b,
          num_tiles, bmm, t_off, n_half):
    C = xf.shape[1]
    nblk = n_half // bmm + num_tiles
    pad_n = nblk * bmm

    oh, pos2, cnt2, x16, idx2 = _router(xf, gate_W, gate_b, num_tiles, bmm,
                                        t_off=t_off, n_half=n_half)
    pos = pos2[:, 0]
    counts = cnt2[0, :num_tiles]

    caps = (counts + bmm - 1) // bmm
    starts = jnp.concatenate([jnp.zeros((1,), jnp.int32),
                              jnp.cumsum(caps)[:-1].astype(jnp.int32)])
    blk_ids = jnp.arange(nblk, dtype=jnp.int32)
    bexp = jnp.sum(blk_ids[None, :] >= starts[1:, None], axis=0,
                   dtype=jnp.int32)
    bexp = jnp.concatenate([bexp, jnp.sum(caps, dtype=jnp.int32)[None]])

    xs_w = _dispatch(x16, pos, pad_n)
    yc_sorted = _grouped_mm(xs_w, up_W, up_b, down_W, down_b, bexp,
                            num_tiles, bmm)
    yc = _combine(yc_sorted, pos, n_half)
    return yc, idx2, oh[:, :num_tiles]


def kernel(x, up_W, up_b, down_W, down_b, gate_W, gate_b):
    Bb, Tt, C = x.shape
    N = Bb * Tt
    num_tiles = gate_W.shape[0]
    BMM = 512
    NH = N // 2

    xf = x.reshape(N, C)

    yc_a, idx_a, gate_a = _half(xf, up_W, up_b, down_W, down_b,
                                gate_W, gate_b, num_tiles, BMM, 0, NH)
    yc_b, idx_b, gate_b_ = _half(xf, up_W, up_b, down_W, down_b,
                                 gate_W, gate_b, num_tiles, BMM,
                                 NH // 512, NH)

    half_blocks = NH // 1024
    out_a = _expand(yc_a, idx_a, C, blk_off=0, n_full=N)
    out = _expand(yc_b, idx_b, C, blk_off=half_blocks, n_full=N, prev=out_a)

    gate_out = jnp.concatenate([gate_a, gate_b_], axis=0)
    return (out.reshape(Bb, Tt, C),
            gate_out.reshape(Bb, Tt, num_tiles))


# speedup vs baseline: 1.0120x; 1.0120x over previous
import functools

import jax
import jax.numpy as jnp
from jax import lax
from jax.experimental import pallas as pl
from jax.experimental.pallas import tpu as pltpu
from jax.experimental.pallas import tpu_sc as plsc

LANES = 128


def _router_body(x_ref, gW_ref, gb_ref, oh_ref, pos_ref, cnt_ref, x16_ref,
                 idx_ref, oh_all, run_s, base_s,
                 *, num_tiles, num_t, bt_blk, bmm):
    p = pl.program_id(0)
    t = pl.program_id(1)
    bt = oh_ref.shape[0]
    hp = jax.lax.Precision.HIGHEST
    ch = x_ref.shape[1] // 2
    xl = x_ref[:, :ch].astype(jnp.bfloat16).astype(jnp.float32)
    xr = x_ref[:, ch:].astype(jnp.bfloat16).astype(jnp.float32)
    ul = jax.lax.bitcast_convert_type(xl, jnp.uint32)
    ur = jax.lax.bitcast_convert_type(xr, jnp.uint32)
    w = (ul & jnp.uint32(0xFFFF0000)) | (ur >> 16)
    x16_ref[...] = jax.lax.bitcast_convert_type(w, jnp.float32)

    @pl.when(p == 0)
    def _pass0():
        @pl.when(t == 0)
        def _init():
            run_s[...] = jnp.zeros_like(run_s)

        logits = jax.lax.dot_general(
            x_ref[...], gW_ref[...], (((1,), (1,)), ((), ())),
            preferred_element_type=jnp.float32) + gb_ref[...]
        cols = jax.lax.broadcasted_iota(jnp.int32, (bt, LANES), 1)
        logits = jnp.where(cols < num_tiles, logits, jnp.float32(-3e38))
        m = jnp.max(logits, axis=1, keepdims=True)
        first = jnp.min(jnp.where(logits >= m, cols, jnp.int32(LANES)),
                        axis=1, keepdims=True)
        oh = (cols == first).astype(jnp.float32)
        oh_ref[...] = oh
        idx_ref[...] = jnp.broadcast_to(first, (bt, LANES))
        pos_ref[...] = jnp.zeros((bt, LANES), jnp.int32)
        oh_all[pl.ds(t * bt_blk, bt_blk), :] = oh
        run_s[...] += jnp.sum(oh, axis=0, keepdims=True)

        @pl.when(t == num_t - 1)
        def _fin():
            cnt = run_s[...].astype(jnp.int32)
            cnt_ref[...] = cnt
            capt = (((cnt + (bmm - 1)) // bmm) * bmm).astype(jnp.float32)
            r = jax.lax.broadcasted_iota(jnp.int32, (LANES, LANES), 0)
            c = jax.lax.broadcasted_iota(jnp.int32, (LANES, LANES), 1)
            triu = (r < c).astype(jnp.float32)
            base_s[...] = jax.lax.dot_general(
                capt, triu, (((1,), (0,)), ((), ())),
                preferred_element_type=jnp.float32, precision=hp)

    @pl.when(p == 1)
    def _pass1():
        @pl.when(t == 0)
        def _init():
            run_s[...] = base_s[...]

        oh = oh_all[pl.ds(t * bt_blk, bt_blk), :]
        oh_ref[...] = oh
        cols1 = jax.lax.broadcasted_iota(jnp.int32, (bt, LANES), 1)
        idx_ref[...] = jnp.broadcast_to(
            jnp.sum(oh * cols1.astype(jnp.float32), axis=1,
                    keepdims=True).astype(jnp.int32), (bt, LANES))
        r = jax.lax.broadcasted_iota(jnp.int32, (bt, bt), 0)
        c = jax.lax.broadcasted_iota(jnp.int32, (bt, bt), 1)
        tri = (r > c).astype(jnp.float32)
        ecs = jax.lax.dot_general(tri, oh, (((1,), (0,)), ((), ())),
                                  preferred_element_type=jnp.float32,
                                  precision=hp)
        pos = jnp.sum((ecs + run_s[...]) * oh, axis=1, keepdims=True)
        pos_ref[...] = jnp.broadcast_to(pos.astype(jnp.int32), (bt, LANES))
        run_s[...] += jnp.sum(oh, axis=0, keepdims=True)


def _router(xf, gate_W, gate_b, num_tiles, bmm, t_off=0, n_half=None):
    C = xf.shape[1]
    N = n_half if n_half is not None else xf.shape[0]
    BT = 512
    num_t = N // BT
    gW = jnp.zeros((LANES, C), jnp.float32).at[:num_tiles].set(gate_W)
    gb = jnp.zeros((1, LANES), jnp.float32).at[0, :num_tiles].set(gate_b)
    return pl.pallas_call(
        functools.partial(_router_body, num_tiles=num_tiles, num_t=num_t,
                          bt_blk=BT, bmm=bmm),
        grid=(2, num_t),
        in_specs=[
            pl.BlockSpec((BT, C), lambda p, t: (t_off + t * (1 - p), 0)),
            pl.BlockSpec((LANES, C), lambda p, t: (0, 0)),
            pl.BlockSpec((1, LANES), lambda p, t: (0, 0)),
        ],
        out_specs=[
            pl.BlockSpec((BT, LANES), lambda p, t: (t, 0)),
            pl.BlockSpec((BT, LANES), lambda p, t: (t, 0)),
            pl.BlockSpec((1, LANES), lambda p, t: (0, 0)),
            pl.BlockSpec((BT, C // 2), lambda p, t: (t * (1 - p), 0)),
            pl.BlockSpec((BT, LANES), lambda p, t: (t, 0)),
        ],
        out_shape=[
            jax.ShapeDtypeStruct((N, LANES), jnp.float32),
            jax.ShapeDtypeStruct((N, LANES), jnp.int32),
            jax.ShapeDtypeStruct((1, LANES), jnp.int32),
            jax.ShapeDtypeStruct((N, C // 2), jnp.float32),
            jax.ShapeDtypeStruct((N, LANES), jnp.int32),
        ],
        scratch_shapes=[
            pltpu.VMEM((N, LANES), jnp.float32),
            pltpu.VMEM((1, LANES), jnp.float32),
            pltpu.VMEM((1, LANES), jnp.float32),
        ],
    )(xf, gW, gb)


def _mm_body(bexp_ref, xs_ref, upW_ref, upb_ref, dW_ref, db_ref, y_ref,
             *, out_tile, nblk):
    i = pl.program_id(0)
    e = bexp_ref[i]
    used = bexp_ref[nblk]

    @pl.when(i < used)
    def _compute():
        u = jax.lax.bitcast_convert_type(xs_ref[...], jnp.uint32)
        xl = jax.lax.bitcast_convert_type(u & jnp.uint32(0xFFFF0000),
                                          jnp.float32)
        xr = jax.lax.bitcast_convert_type(u << 16, jnp.float32)
        x = jnp.concatenate([xl, xr], axis=1).astype(jnp.bfloat16)
        h = jax.lax.dot_general(x, upW_ref[0].astype(jnp.bfloat16),
                                (((1,), (1,)), ((), ())),
                                preferred_element_type=jnp.float32)
        h = jnp.maximum(h + upb_ref[0], 0.0).astype(jnp.bfloat16)
        y = jax.lax.dot_general(h, dW_ref[0].astype(jnp.bfloat16),
                                (((1,), (1,)), ((), ())),
                                preferred_element_type=jnp.float32)
        y_ref[...] = y + db_ref[0]


def _grouped_mm(xs, up_W, up_b, down_W, down_b, bexp, num_tiles, bmm):
    PAD_N = xs.shape[0]
    C = up_W.shape[1]
    d_ff = up_W.shape[0]
    ftile = d_ff // num_tiles
    out_tile = C // num_tiles
    nblk = PAD_N // bmm
    upW4 = up_W.reshape(num_tiles, ftile, C)
    upb3 = up_b.reshape(num_tiles, 1, ftile)
    dW4 = down_W.reshape(num_tiles, out_tile, num_tiles * ftile)
    db3 = down_b.reshape(num_tiles, 1, out_tile)
    grid_spec = pltpu.PrefetchScalarGridSpec(
        num_scalar_prefetch=1,
        grid=(nblk,),
        in_specs=[
            pl.BlockSpec((bmm, C // 2), lambda i, b: (i, 0)),
            pl.BlockSpec((1, ftile, C), lambda i, b: (b[i], 0, 0)),
            pl.BlockSpec((1, 1, ftile), lambda i, b: (b[i], 0, 0)),
            pl.BlockSpec((1, out_tile, ftile),
                         lambda i, b: (b[i], 0, b[i])),
            pl.BlockSpec((1, 1, out_tile), lambda i, b: (b[i], 0, 0)),
        ],
        out_specs=pl.BlockSpec((bmm, out_tile), lambda i, b: (i, 0)),
    )
    return pl.pallas_call(
        functools.partial(_mm_body, out_tile=out_tile, nblk=nblk),
        grid_spec=grid_spec,
        out_shape=jax.ShapeDtypeStruct((PAD_N, out_tile), jnp.float32),
    )(bexp, xs, upW4, upb3, dW4, db3)


def _expand_body(yc_ref, idx_ref, out_ref, *, out_tile):
    y = yc_ref[...]
    e = idx_ref[:, :1]
    reps = out_ref.shape[1] // out_tile
    ytile = jnp.concatenate([y] * reps, axis=1)
    ocols = jax.lax.broadcasted_iota(jnp.int32, ytile.shape, 1)
    out_ref[...] = jnp.where((ocols // out_tile) == e, ytile, 0.0)


def _expand_body_alias(yc_ref, idx_ref, prev_ref, out_ref, *, out_tile):
    del prev_ref
    _expand_body(yc_ref, idx_ref, out_ref, out_tile=out_tile)


def _expand(yc, idx2, C, blk_off=0, n_full=None, prev=None):
    N, out_tile = yc.shape
    n_full = n_full if n_full is not None else N
    BT = 1024
    in_specs = [
        pl.BlockSpec((BT, out_tile), lambda t: (t, 0)),
        pl.BlockSpec((BT, LANES), lambda t: (t, 0)),
    ]
    args = [yc, idx2]
    kwargs = {}
    body = functools.partial(_expand_body, out_tile=out_tile)
    if prev is not None:
        in_specs.append(pl.BlockSpec(memory_space=pltpu.HBM))
        args.append(prev)
        kwargs["input_output_aliases"] = {2: 0}
        body = functools.partial(_expand_body_alias, out_tile=out_tile)
    return pl.pallas_call(
        body,
        grid=(N // BT,),
        in_specs=in_specs,
        out_specs=pl.BlockSpec((BT, C), lambda t: (blk_off + t, 0)),
        out_shape=jax.ShapeDtypeStruct((n_full, C), jnp.float32),
        **kwargs,
    )(*args)


def _sc_mesh():
    info = plsc.get_sparse_core_info()
    return plsc.VectorSubcoreMesh(core_axis_name="c", subcore_axis_name="s"), \
        info.num_cores, info.num_subcores


def _dispatch(xf, pos, pad_n):
    N, C = xf.shape
    mesh, nc, ns = _sc_mesh()
    per_w = N // (nc * ns)
    CH = 16
    nchunk = per_w // CH

    @functools.partial(
        pl.kernel, mesh=mesh,
        out_type=jax.ShapeDtypeStruct((pad_n, C), xf.dtype),
        scratch_types=[
            pltpu.VMEM((CH,), jnp.int32),
            pltpu.VMEM((CH, C), xf.dtype),
            pltpu.SemaphoreType.DMA,
        ],
    )
    def k(xf_h, pos_h, xs_h, pos_v, xbuf, sem):
        wid = lax.axis_index("s") * nc + lax.axis_index("c")

        def chunk(j, _):
            n0 = wid * per_w + j * CH
            pltpu.sync_copy(pos_h.at[pl.ds(n0, CH)], pos_v)
            pltpu.sync_copy(xf_h.at[pl.ds(n0, CH)], xbuf)
            pltpu.async_copy(xbuf, xs_h.at[pos_v], sem).wait()
            return ()

        lax.fori_loop(0, nchunk, chunk, (), unroll=False)

    return k(xf, pos)


def _combine(y_full, pos, n_out):
    PAD_N, C = y_full.shape
    mesh, nc, ns = _sc_mesh()
    per_w = n_out // (nc * ns)
    CH = 16
    nchunk = per_w // CH

    @functools.partial(
        pl.kernel, mesh=mesh,
        out_type=jax.ShapeDtypeStruct((n_out, C), jnp.float32),
        scratch_types=[
            pltpu.VMEM((CH,), jnp.int32),
            pltpu.VMEM((CH, C), jnp.float32),
            pltpu.SemaphoreType.DMA,
        ],
    )
    def k(y_h, pos_h, out_h, pos_v, ybuf, sem):
        wid = lax.axis_index("s") * nc + lax.axis_index("c")

        def chunk(j, _):
            n0 = wid * per_w + j * CH
            pltpu.sync_copy(pos_h.at[pl.ds(n0, CH)], pos_v)
            pltpu.async_copy(y_h.at[pos_v], ybuf, sem).wait()
            pltpu.sync_copy(ybuf, out_h.at[pl.ds(n0, CH)])
            return ()

        lax.fori_loop(0, nchunk, chunk, (), unroll=False)

    return k(y_full, pos)


def _half(xf, up_W, up_b, down_W, down_b, gate_W, gate_b,
          num_tiles, bmm, t_off, n_half):
    C = xf.shape[1]
    nblk = n_half // bmm + num_tiles
    pad_n = nblk * bmm

    oh, pos2, cnt2, x16, idx2 = _router(xf, gate_W, gate_b, num_tiles, bmm,
                                        t_off=t_off, n_half=n_half)
    pos = pos2[:, 0]
    counts = cnt2[0, :num_tiles]

    caps = (counts + bmm - 1) // bmm
    starts = jnp.concatenate([jnp.zeros((1,), jnp.int32),
                              jnp.cumsum(caps)[:-1].astype(jnp.int32)])
    blk_ids = jnp.arange(nblk, dtype=jnp.int32)
    bexp = jnp.sum(blk_ids[None, :] >= starts[1:, None], axis=0,
                   dtype=jnp.int32)
    bexp = jnp.concatenate([bexp, jnp.sum(caps, dtype=jnp.int32)[None]])

    xs_w = _dispatch(x16, pos, pad_n)
    yc_sorted = _grouped_mm(xs_w, up_W, up_b, down_W, down_b, bexp,
                            num_tiles, bmm)
    yc = _combine(yc_sorted, pos, n_half)
    return yc, idx2, oh[:, :num_tiles]


def kernel(x, up_W, up_b, down_W, down_b, gate_W, gate_b):
    Bb, Tt, C = x.shape
    N = Bb * Tt
    num_tiles = gate_W.shape[0]
    BMM = 512
    NH = N // 2

    xf = x.reshape(N, C)

    yc_a, idx_a, gate_a = _half(xf, up_W, up_b, down_W, down_b,
                                gate_W, gate_b, num_tiles, BMM, 0, NH)
    yc_b, idx_b, gate_b_ = _half(xf, up_W, up_b, down_W, down_b,
                                 gate_W, gate_b, num_tiles, BMM,
                                 NH // 512, NH)

    half_blocks = NH // 1024
    out_a = _expand(yc_a, idx_a, C, blk_off=0, n_full=N)
    out = _expand(yc_b, idx_b, C, blk_off=half_blocks, n_full=N, prev=out_a)

    gate_out = jnp.concatenate([gate_a, gate_b_], axis=0)
    return (out.reshape(Bb, Tt, C),
            gate_out.reshape(Bb, Tt, num_tiles))
